# baseline (speedup 1.0000x reference)
import functools

import jax
import jax.numpy as jnp
from jax import lax
from jax.experimental import pallas as pl
from jax.experimental.pallas import tpu as pltpu
from jax.experimental.pallas import tpu_sc as plsc

_NORMALIZATION_FACTOR = 9.48933
_N = 50000
_L = 16
_NT = 16
_VECS = 196
_E = _VECS * _L
_LAST_BASE = _N - _E

_mesh = plsc.VectorSubcoreMesh(
    core_axis_name="c", subcore_axis_name="s", num_cores=1
)


@functools.partial(
    pl.kernel,
    out_type=jax.ShapeDtypeStruct((_N,), jnp.float32),
    mesh=_mesh,
    scratch_types=[
        pltpu.VMEM((_L,), jnp.float32),
        pltpu.VMEM((_E,), jnp.int32),
        pltpu.VMEM((_E,), jnp.float32),
        pltpu.SemaphoreType.DMA,
        pltpu.SemaphoreType.DMA,
        pltpu.SemaphoreType.DMA,
        pltpu.SemaphoreType.DMA,
    ],
)
def _fixed_charge_sc(
    an_hbm, table_hbm, out_hbm, table_v, idx_v, out_v, sem_t, sem_a, sem_b, sem_oa
):
    wid = lax.axis_index("s")
    base = jnp.where(wid == _NT - 1, _LAST_BASE, wid * _E)
    half = _E // 2

    ca = pltpu.async_copy(
        an_hbm.at[pl.ds(base, half)], idx_v.at[pl.ds(0, half)], sem_a
    )
    cb = pltpu.async_copy(
        an_hbm.at[pl.ds(base + half, half)], idx_v.at[pl.ds(half, half)], sem_b
    )
    ct = pltpu.async_copy(table_hbm, table_v, sem_t)
    ct.wait()
    tv = table_v[...] * _NORMALIZATION_FACTOR

    dnums = lax.GatherDimensionNumbers(
        offset_dims=(), collapsed_slice_dims=(0,), start_index_map=(0,)
    )

    def lookup(lo, hi):
        def body(i, _):
            iv = idx_v[pl.ds(i * _L, _L)]
            out_v[pl.ds(i * _L, _L)] = lax.gather(
                tv,
                iv[:, None],
                dnums,
                slice_sizes=(1,),
                mode=lax.GatherScatterMode.PROMISE_IN_BOUNDS,
            )
            return 0

        lax.fori_loop(lo, hi, body, 0, unroll=8)

    ca.wait()
    lookup(0, _VECS // 2)
    coa = pltpu.async_copy(
        out_v.at[pl.ds(0, half)], out_hbm.at[pl.ds(base, half)], sem_oa
    )
    cb.wait()
    lookup(_VECS // 2, _VECS)
    pltpu.sync_copy(
        out_v.at[pl.ds(half, half)], out_hbm.at[pl.ds(base + half, half)]
    )
    coa.wait()


def kernel(atomic_numbers, charge_table):
    table16 = jnp.zeros((_L,), jnp.float32).at[:10].set(charge_table)
    an = atomic_numbers.astype(jnp.int32)
    out = _fixed_charge_sc(an, table16)
    return out[:, None]


# speedup vs baseline: 1.0534x; 1.0534x over previous
import functools

import jax
import jax.numpy as jnp
from jax import lax
from jax.experimental import pallas as pl
from jax.experimental.pallas import tpu as pltpu
from jax.experimental.pallas import tpu_sc as plsc

_NORMALIZATION_FACTOR = 9.48933
_N = 50000
_NTYPES = 10
_L = 16
_NT = 16
_VECS = 196
_E = _VECS * _L
_LAST_BASE = _N - _E
_Q = _E // 4

_mesh = plsc.VectorSubcoreMesh(
    core_axis_name="c", subcore_axis_name="s", num_cores=1
)


@functools.partial(
    pl.kernel,
    out_type=jax.ShapeDtypeStruct((_N,), jnp.float32),
    mesh=_mesh,
    scratch_types=[
        pltpu.VMEM((_L,), jnp.float32),
        pltpu.VMEM((_E,), jnp.int32),
        pltpu.VMEM((_E,), jnp.float32),
        pltpu.SemaphoreType.DMA,
        pltpu.SemaphoreType.DMA,
        pltpu.SemaphoreType.DMA,
        pltpu.SemaphoreType.DMA,
    ],
)
def _fixed_charge_sc(
    an_hbm, table_hbm, out_hbm, table_v, idx_v, out_v, sem_t, sem_a, sem_b, sem_o
):
    wid = lax.axis_index("s")
    base = jnp.where(wid == _NT - 1, _LAST_BASE, wid * _E)
    half = _E // 2

    ca = pltpu.async_copy(
        an_hbm.at[pl.ds(base, half)], idx_v.at[pl.ds(0, half)], sem_a
    )
    cb = pltpu.async_copy(
        an_hbm.at[pl.ds(base + half, half)], idx_v.at[pl.ds(half, half)], sem_b
    )
    ct = pltpu.async_copy(table_hbm, table_v.at[pl.ds(0, _NTYPES)], sem_t)
    ct.wait()
    tv = table_v[...] * _NORMALIZATION_FACTOR

    dnums = lax.GatherDimensionNumbers(
        offset_dims=(), collapsed_slice_dims=(0,), start_index_map=(0,)
    )

    def lookup(lo, hi):
        for i in range(lo, hi):
            iv = idx_v[pl.ds(i * _L, _L)]
            out_v[pl.ds(i * _L, _L)] = lax.gather(
                tv,
                iv[:, None],
                dnums,
                slice_sizes=(1,),
                mode=lax.GatherScatterMode.PROMISE_IN_BOUNDS,
            )

    outs = []
    ca.wait()
    for q in range(2):
        lookup(q * (_VECS // 4), (q + 1) * (_VECS // 4))
        outs.append(
            pltpu.async_copy(
                out_v.at[pl.ds(q * _Q, _Q)], out_hbm.at[pl.ds(base + q * _Q, _Q)], sem_o
            )
        )
    cb.wait()
    for q in range(2, 4):
        lookup(q * (_VECS // 4), (q + 1) * (_VECS // 4))
        outs.append(
            pltpu.async_copy(
                out_v.at[pl.ds(q * _Q, _Q)], out_hbm.at[pl.ds(base + q * _Q, _Q)], sem_o
            )
        )
    for c in outs:
        c.wait()


def kernel(atomic_numbers, charge_table):
    an = atomic_numbers.astype(jnp.int32)
    out = _fixed_charge_sc(an, charge_table.astype(jnp.float32))
    return out[:, None]
